# baseline (speedup 1.0000x reference)
import dataclasses
import functools

import jax
import jax.numpy as jnp
from jax import lax
from jax.experimental import pallas as pl
from jax.experimental.pallas import tpu as pltpu
from jax.experimental.pallas import tpu_sc as plsc

D = 128
L = 16
NS = 16
NC = 2
CHUNK = 80
ROWB = 80


def _sc_compiler_params():
    cp = pltpu.CompilerParams()
    if "needs_layout_passes" in pltpu.CompilerParams.__dataclass_fields__:
        cp = dataclasses.replace(cp, needs_layout_passes=False)
    return cp


TBLK = 1000


def _logit_tables_kernel(ent_ref, rel_ref, v02_ref, v1_ref, b1_ref,
                         s02_ref, s1_ref):
    hi = jax.lax.Precision.HIGHEST
    s02_ref[...] = jnp.dot(ent_ref[...], v02_ref[...], precision=hi,
                           preferred_element_type=jnp.float32)
    s1_ref[...] = jnp.dot(rel_ref[...], v1_ref[...], precision=hi,
                          preferred_element_type=jnp.float32) + b1_ref[0, 0]


def _p_tables_kernel(ent_ref, rel_ref, w0t_ref, w1t_ref, w2t_ref, ba_ref,
                     p0_ref, p1_ref, p2_ref):
    hi = jax.lax.Precision.HIGHEST
    p0_ref[...] = jnp.dot(ent_ref[...], w0t_ref[...], precision=hi,
                          preferred_element_type=jnp.float32)
    p1_ref[...] = jnp.dot(rel_ref[...], w1t_ref[...], precision=hi,
                          preferred_element_type=jnp.float32) + ba_ref[...]
    p2_ref[...] = jnp.dot(ent_ref[...], w2t_ref[...], precision=hi,
                          preferred_element_type=jnp.float32)


def _build_tables(ent, rel, w0t, w1t, w2t, ba_row, w2col, ba2):
    n = ent.shape[0]
    f32 = jnp.float32
    row_spec = pl.BlockSpec((TBLK, D), lambda i: (i, 0))
    s_spec = pl.BlockSpec((TBLK, 1), lambda i: (i, 0))
    s2_spec = pl.BlockSpec((TBLK, 2), lambda i: (i, 0))
    full = lambda shape: pl.BlockSpec(shape, lambda i: tuple(0 for _ in shape))
    v02 = jnp.concatenate([w0t @ w2col, w2t @ w2col], axis=1)
    v1 = w1t @ w2col
    s1_bias = (ba_row @ w2col + ba2).reshape(1, 1)
    s02, s1 = pl.pallas_call(
        _logit_tables_kernel,
        grid=(n // TBLK,),
        in_specs=[row_spec, row_spec, full((D, 2)), full((D, 1)),
                  full((1, 1))],
        out_specs=[s2_spec, s_spec],
        out_shape=[
            jax.ShapeDtypeStruct((n, 2), f32),
            jax.ShapeDtypeStruct((n, 1), f32),
        ],
    )(ent, rel, v02, v1, s1_bias)
    p0, p1, p2 = pl.pallas_call(
        _p_tables_kernel,
        grid=(n // TBLK,),
        in_specs=[row_spec, row_spec, full((D, D)), full((D, D)),
                  full((D, D)), full((1, D))],
        out_specs=[row_spec, row_spec, row_spec],
        out_shape=[
            jax.ShapeDtypeStruct((n, D), f32),
            jax.ShapeDtypeStruct((n, D), f32),
            jax.ShapeDtypeStruct((n, D), f32),
        ],
    )(ent, rel, w0t, w1t, w2t, ba_row)
    return p0, p1, p2, s02[:, 0], s1[:, 0], s02[:, 1]


def _finalize_kernel(hs_ref, hr_ref, p0_ref, p1_ref, aux0_ref, aux1_ref,
                     aux2_ref, hent_ref, hrel_ref):
    f32 = jnp.float32
    ebs = jnp.sum(aux0_ref[...], axis=0)[:, None]
    ews = jnp.sum(aux1_ref[...], axis=0)[:, None]
    cnt = jnp.sum(aux2_ref[...], axis=0)[:, None]
    hs = hs_ref[...] + p0_ref[...] * ebs
    hr = hr_ref[...] + p1_ref[...] * ews
    den = jnp.where(ebs == 0, f32(1e-12), ebs)
    hent_ref[...] = jnp.maximum(hs / den, f32(0.0))
    hrel_ref[...] = jnp.maximum(hr / jnp.maximum(cnt, f32(1.0)), f32(0.0))


def _finalize(hs, hr, p0, p1, aux0, aux1, aux2):
    f32 = jnp.float32
    return pl.pallas_call(
        _finalize_kernel,
        out_shape=[
            jax.ShapeDtypeStruct(hs.shape, f32),
            jax.ShapeDtypeStruct(hr.shape, f32),
        ],
    )(hs, hr, p0, p1, aux0, aux1, aux2)


def _make_sc_weights(n_ent, n_rel, n_edges):
    nw = NC * NS
    epw = n_edges // nw
    n_chunks = epw // CHUNK
    f32 = jnp.float32
    mesh = plsc.VectorSubcoreMesh(core_axis_name="c", subcore_axis_name="s")

    @functools.partial(
        pl.kernel,
        compiler_params=_sc_compiler_params(),
        out_type=[
            jax.ShapeDtypeStruct((n_edges,), f32),
            jax.ShapeDtypeStruct((nw, n_ent), f32),
            jax.ShapeDtypeStruct((nw, n_rel), f32),
            jax.ShapeDtypeStruct((nw, n_rel), f32),
        ],
        mesh=mesh,
        scratch_types=[
            pltpu.VMEM((CHUNK,), jnp.int32),
            pltpu.VMEM((CHUNK,), jnp.int32),
            pltpu.VMEM((CHUNK,), jnp.int32),
            pltpu.VMEM((CHUNK,), jnp.int32),
            pltpu.VMEM((CHUNK,), jnp.int32),
            pltpu.VMEM((CHUNK,), jnp.int32),
            pltpu.VMEM((CHUNK,), f32),
            pltpu.VMEM((CHUNK,), f32),
            pltpu.VMEM((n_ent,), f32),
            pltpu.VMEM((n_rel,), f32),
            pltpu.VMEM((n_ent,), f32),
            pltpu.VMEM((n_ent,), f32),
            pltpu.VMEM((n_rel,), f32),
            pltpu.VMEM((n_rel,), f32),
            pltpu.SemaphoreType.DMA,
            pltpu.SemaphoreType.DMA,
            pltpu.SemaphoreType.DMA,
            pltpu.SemaphoreType.DMA,
        ],
    )
    def sc_weights(t0_hbm, t1_hbm, t2_hbm, s0_hbm, s1_hbm, s2_hbm,
                   e_hbm, aux0_hbm, aux1_hbm, aux2_hbm,
                   i0a, i1a, i2a, i0b, i1b, i2b, ea, eb,
                   s0t, s1t, s2t, a0l, a1l, a2l,
                   semia, semib, semwa, semwb):
        c = lax.axis_index("c")
        s = lax.axis_index("s")
        wid = c * NS + s
        zero16 = jnp.zeros((L,), f32)
        one16 = jnp.ones((L,), f32)
        ebase = wid * epw

        bufs = [(i0a, i1a, i2a, ea, semia, semwa),
                (i0b, i1b, i2b, eb, semib, semwb)]

        def idx_descs(ci, B):
            i0v, i1v, i2v, _, semi, _ = bufs[B]
            off = ebase + ci * CHUNK
            return [
                pltpu.make_async_copy(t0_hbm.at[pl.ds(off, CHUNK)], i0v, semi),
                pltpu.make_async_copy(t1_hbm.at[pl.ds(off, CHUNK)], i1v, semi),
                pltpu.make_async_copy(t2_hbm.at[pl.ds(off, CHUNK)], i2v, semi),
            ]

        def wb_desc(ci, B):
            _, _, _, e80, _, semw = bufs[B]
            off = ebase + ci * CHUNK
            return pltpu.make_async_copy(e80, e_hbm.at[pl.ds(off, CHUNK)], semw)

        pltpu.sync_copy(s0_hbm, s0t)
        pltpu.sync_copy(s1_hbm, s1t)
        pltpu.sync_copy(s2_hbm, s2t)

        @pl.loop(0, n_ent // L)
        def _(q):
            a0l[pl.ds(q * L, L)] = zero16

        @pl.loop(0, n_rel // L)
        def _(q):
            a1l[pl.ds(q * L, L)] = zero16
            a2l[pl.ds(q * L, L)] = zero16

        def compute_chunk(B):
            i0v, i1v, i2v, e80, _, _ = bufs[B]

            @pl.loop(0, CHUNK // L)
            def _(q):
                sl = pl.ds(q * L, L)
                a0 = i0v[sl]
                a1 = i1v[sl]
                a2 = i2v[sl]
                d = (plsc.load_gather(s0t, [a0]) +
                     plsc.load_gather(s1t, [a1]) +
                     plsc.load_gather(s2t, [a2]))
                lk = jnp.where(d >= 0, d, d * f32(0.01))
                e = jnp.exp(-lk)
                e80[sl] = e
                plsc.addupdate_scatter(a0l, [a0], e)
                plsc.addupdate_scatter(a1l, [a1], e)
                plsc.addupdate_scatter(a2l, [a1], one16)

        for dsc in idx_descs(0, 0):
            dsc.start()
        for dsc in idx_descs(1, 1):
            dsc.start()

        @pl.loop(0, n_chunks // 2)
        def _(gg):
            for b in (0, 1):
                ci = gg * 2 + b
                for dsc in idx_descs(ci, b):
                    dsc.wait()

                @pl.when(gg >= 1)
                def _():
                    wb_desc(ci - 2, b).wait()

                compute_chunk(b)
                wb_desc(ci, b).start()

                @pl.when(ci + 2 < n_chunks)
                def _():
                    for dsc in idx_descs(ci + 2, b):
                        dsc.start()

        if n_chunks % 2 == 1:
            last = n_chunks - 1
            for dsc in idx_descs(last, 0):
                dsc.wait()
            wb_desc(last - 2, 0).wait()
            compute_chunk(0)
            wb_desc(last, 0).start()
            wb_desc(last - 1, 1).wait()
            wb_desc(last, 0).wait()
        else:
            wb_desc(n_chunks - 2, 0).wait()
            wb_desc(n_chunks - 1, 1).wait()

        pltpu.sync_copy(a0l, aux0_hbm.at[wid])
        pltpu.sync_copy(a1l, aux1_hbm.at[wid])
        pltpu.sync_copy(a2l, aux2_hbm.at[wid])

    return sc_weights


def _make_sc_agg(n_ent, n_rel, n_edges):
    edges_per_tile = n_edges // NS
    n_chunks = edges_per_tile // CHUNK
    assert n_chunks % 2 == 0
    nrb_ent = n_ent // ROWB
    nrb_rel = n_rel // ROWB
    n_max = max(n_ent, n_rel)
    f32 = jnp.float32
    mesh = plsc.VectorSubcoreMesh(core_axis_name="c", subcore_axis_name="s")

    idx_t = pltpu.VMEM((CHUNK,), jnp.int32)
    row_t = pltpu.VMEM((CHUNK, D), f32)
    e_t = pltpu.VMEM((CHUNK + L,), f32)

    @functools.partial(
        pl.kernel,
        compiler_params=_sc_compiler_params(),
        out_type=[
            jax.ShapeDtypeStruct((n_ent, D), f32),
            jax.ShapeDtypeStruct((n_rel, D), f32),
        ],
        mesh=mesh,
        scratch_types=(
            [idx_t] * 6 + [idx_t] * 2 + [row_t] * 4 + [e_t] * 2 +
            [pltpu.VMEM_SHARED((n_max, D), f32)] +
            [pltpu.SemaphoreType.DMA] * 6
        ),
    )
    def sc_agg(t0_hbm, t1_hbm, t2_hbm, e_hbm, p0_hbm, p1_hbm, p2_hbm,
               hent_hbm, hrel_hbm,
               i0a, i1a, i2a, i0b, i1b, i2b, ka, kb, gaa, gba, gab, gbb,
               ea, eb, accum,
               semia, semib, semga, semgb, semsa, semsb):
        c = lax.axis_index("c")
        s = lax.axis_index("s")
        zero16 = jnp.zeros((L,), f32)
        ebase = s * edges_per_tile

        bufs = [
            (i0a, i1a, i2a, ka, gaa, gba, ea, semia, semga, semsa),
            (i0b, i1b, i2b, kb, gab, gbb, eb, semib, semgb, semsb),
        ]

        def idx_descs(ci, B):
            i0v, i1v, i2v, _, _, _, e80, semi, _, _ = bufs[B]
            off = ebase + ci * CHUNK
            return [
                pltpu.make_async_copy(t0_hbm.at[pl.ds(off, CHUNK)], i0v, semi),
                pltpu.make_async_copy(t1_hbm.at[pl.ds(off, CHUNK)], i1v, semi),
                pltpu.make_async_copy(t2_hbm.at[pl.ds(off, CHUNK)], i2v, semi),
                pltpu.make_async_copy(e_hbm.at[pl.ds(off, CHUNK)],
                                      e80.at[pl.ds(0, CHUNK)], semi),
            ]

        def issue_idx(ci, B):
            for dsc in idx_descs(ci, B):
                dsc.start()

        def wait_idx(ci, B):
            for dsc in idx_descs(ci, B):
                dsc.wait()

        def gather_descs(core, B):
            i0v, i1v, i2v, _, gav, gbv, _, _, semg, _ = bufs[B]
            if core == 0:
                return [pltpu.make_async_copy(p1_hbm.at[i1v], gav, semg),
                        pltpu.make_async_copy(p2_hbm.at[i2v], gbv, semg)]
            return [pltpu.make_async_copy(p0_hbm.at[i0v], gav, semg),
                    pltpu.make_async_copy(p2_hbm.at[i2v], gbv, semg)]

        def for_core(fn):
            @pl.when(c == 0)
            def _():
                fn(0)

            @pl.when(c == 1)
            def _():
                fn(1)

        def issue_gather(B):
            for_core(lambda core: [d.start() for d in gather_descs(core, B)])

        def wait_gather(B):
            for_core(lambda core: [d.wait() for d in gather_descs(core, B)])

        def scatter_desc(B):
            _, _, _, kv, gav, _, _, _, _, sems = bufs[B]
            return pltpu.make_async_copy(gav, accum.at[kv], sems)

        def copy_key(B):
            i0v, i1v, _, kv, _, _, _, _, _, _ = bufs[B]

            @pl.when(c == 0)
            def _():
                for q in range(CHUNK // L):
                    sl = pl.ds(q * L, L)
                    kv[sl] = i0v[sl]

            @pl.when(c == 1)
            def _():
                for q in range(CHUNK // L):
                    sl = pl.ds(q * L, L)
                    kv[sl] = i1v[sl]

        def compute(B):
            _, _, _, _, gav, gbv, e80, _, _, _ = bufs[B]

            @pl.loop(0, CHUNK // L)
            def _(q):
                ev = e80[pl.ds(q * L, L)]
                for l in range(L):
                    j = q * L + l
                    es = ev[l]
                    for k in range(D // L):
                        sl = pl.ds(k * L, L)
                        gav[j, sl] = (gav[j, sl] + gbv[j, sl]) * es

        @pl.loop(0, ROWB)
        def _(rr):
            for k in range(D // L):
                gaa[rr, pl.ds(k * L, L)] = zero16

        nrb = jnp.where(c == 0, nrb_ent, nrb_rel)

        @pl.loop(s, nrb, step=NS)
        def _(bi):
            pltpu.sync_copy(gaa, accum.at[pl.ds(bi * ROWB, ROWB)])

        plsc.subcore_barrier()

        issue_idx(0, 0)
        issue_idx(1, 1)
        wait_idx(0, 0)
        issue_gather(0)

        @pl.loop(0, n_chunks // 2)
        def _(gg):
            for b in (0, 1):
                bn = 1 - b
                ci = gg * 2 + b
                wait_gather(b)
                copy_key(b)

                @pl.when(ci + 1 < n_chunks)
                def _():
                    wait_idx(ci + 1, bn)

                    @pl.when(ci >= 1)
                    def _():
                        scatter_desc(bn).wait()
                    issue_gather(bn)

                compute(b)

                @pl.when(ci + 2 < n_chunks)
                def _():
                    issue_idx(ci + 2, b)

                pltpu.async_copy(bufs[b][4], accum.at[bufs[b][3]],
                                 bufs[b][9], add=True)

        scatter_desc(0).wait()
        scatter_desc(1).wait()

        plsc.subcore_barrier()

        @pl.when(c == 0)
        def _():
            @pl.loop(s, nrb_ent, step=NS)
            def _(bi):
                base = bi * ROWB
                pltpu.sync_copy(accum.at[pl.ds(base, ROWB)],
                                hent_hbm.at[pl.ds(base, ROWB)])

        @pl.when(c == 1)
        def _():
            @pl.loop(s, nrb_rel, step=NS)
            def _(bi):
                base = bi * ROWB
                pltpu.sync_copy(accum.at[pl.ds(base, ROWB)],
                                hrel_hbm.at[pl.ds(base, ROWB)])

    return sc_agg


def kernel(triplets, ent_embed, rel_embed, W_a, b_a, W_a2, b_a2):
    n_ent = ent_embed.shape[0]
    n_rel = rel_embed.shape[0]
    n_edges = triplets.shape[0]

    t0 = triplets[:, 0]
    t1 = triplets[:, 1]
    t2 = triplets[:, 2]

    w0t = W_a[:, 0:D].T
    w1t = W_a[:, D:2 * D].T
    w2t = W_a[:, 2 * D:3 * D].T
    ba_row = b_a.reshape(1, D)
    w2col = W_a2.T
    ba2 = b_a2.reshape(1, 1)

    p0, p1, p2, s0, s1, s2 = _build_tables(
        ent_embed, rel_embed, w0t, w1t, w2t, ba_row, w2col, ba2)

    sc_weights = _make_sc_weights(n_ent, n_rel, n_edges)
    e_all, aux0, aux1, aux2 = sc_weights(t0, t1, t2, s0, s1, s2)

    sc_agg = _make_sc_agg(n_ent, n_rel, n_edges)
    hs, hr = sc_agg(t0, t1, t2, e_all, p0, p1, p2)

    return _finalize(hs, hr, p0, p1, aux0, aux1, aux2)


# speedup vs baseline: 1.5026x; 1.5026x over previous
import dataclasses
import functools

import jax
import jax.numpy as jnp
from jax import lax
from jax.experimental import pallas as pl
from jax.experimental.pallas import tpu as pltpu
from jax.experimental.pallas import tpu_sc as plsc

D = 128
L = 16
NS = 16
NC = 2
CHUNK = 80
ROWB = 80


def _sc_compiler_params():
    cp = pltpu.CompilerParams()
    if "needs_layout_passes" in pltpu.CompilerParams.__dataclass_fields__:
        cp = dataclasses.replace(cp, needs_layout_passes=False)
    return cp


TBLK = 1000


def _logit_tables_kernel(ent_ref, rel_ref, v02_ref, v1_ref, b1_ref,
                         s02_ref, s1_ref):
    hi = jax.lax.Precision.HIGHEST
    s02_ref[...] = jnp.dot(ent_ref[...], v02_ref[...], precision=hi,
                           preferred_element_type=jnp.float32)
    s1_ref[...] = jnp.dot(rel_ref[...], v1_ref[...], precision=hi,
                          preferred_element_type=jnp.float32) + b1_ref[0, 0]


def _p_tables_kernel(ent_ref, rel_ref, w0t_ref, w1t_ref, w2t_ref, ba_ref,
                     p0_ref, p1_ref, p2_ref):
    hi = jax.lax.Precision.HIGHEST
    p0_ref[...] = jnp.dot(ent_ref[...], w0t_ref[...], precision=hi,
                          preferred_element_type=jnp.float32)
    p1_ref[...] = jnp.dot(rel_ref[...], w1t_ref[...], precision=hi,
                          preferred_element_type=jnp.float32) + ba_ref[...]
    p2_ref[...] = jnp.dot(ent_ref[...], w2t_ref[...], precision=hi,
                          preferred_element_type=jnp.float32)


def _build_tables(ent, rel, w0t, w1t, w2t, ba_row, w2col, ba2):
    n = ent.shape[0]
    f32 = jnp.float32
    row_spec = pl.BlockSpec((TBLK, D), lambda i: (i, 0))
    s_spec = pl.BlockSpec((TBLK, 1), lambda i: (i, 0))
    s2_spec = pl.BlockSpec((TBLK, 2), lambda i: (i, 0))
    full = lambda shape: pl.BlockSpec(shape, lambda i: tuple(0 for _ in shape))
    v02 = jnp.concatenate([w0t @ w2col, w2t @ w2col], axis=1)
    v1 = w1t @ w2col
    s1_bias = (ba_row @ w2col + ba2).reshape(1, 1)
    s02, s1 = pl.pallas_call(
        _logit_tables_kernel,
        grid=(n // TBLK,),
        in_specs=[row_spec, row_spec, full((D, 2)), full((D, 1)),
                  full((1, 1))],
        out_specs=[s2_spec, s_spec],
        out_shape=[
            jax.ShapeDtypeStruct((n, 2), f32),
            jax.ShapeDtypeStruct((n, 1), f32),
        ],
    )(ent, rel, v02, v1, s1_bias)
    p0, p1, p2 = pl.pallas_call(
        _p_tables_kernel,
        grid=(n // TBLK,),
        in_specs=[row_spec, row_spec, full((D, D)), full((D, D)),
                  full((D, D)), full((1, D))],
        out_specs=[row_spec, row_spec, row_spec],
        out_shape=[
            jax.ShapeDtypeStruct((n, D), f32),
            jax.ShapeDtypeStruct((n, D), f32),
            jax.ShapeDtypeStruct((n, D), f32),
        ],
    )(ent, rel, w0t, w1t, w2t, ba_row)
    return p0, p1, p2, s02[:, 0], s1[:, 0], s02[:, 1]


def _finalize_kernel(hs_ref, hr_ref, p0_ref, p1_ref, aux0_ref, aux1_ref,
                     aux2_ref, hent_ref, hrel_ref):
    f32 = jnp.float32
    ebs = jnp.sum(aux0_ref[...], axis=0)[:, None]
    ews = jnp.sum(aux1_ref[...], axis=0)[:, None]
    cnt = jnp.sum(aux2_ref[...], axis=0)[:, None]
    hs = hs_ref[...] + p0_ref[...] * ebs
    hr = hr_ref[...] + p1_ref[...] * ews
    den = jnp.where(ebs == 0, f32(1e-12), ebs)
    hent_ref[...] = jnp.maximum(hs / den, f32(0.0))
    hrel_ref[...] = jnp.maximum(hr / jnp.maximum(cnt, f32(1.0)), f32(0.0))


def _finalize(hs, hr, p0, p1, aux0, aux1, aux2):
    f32 = jnp.float32
    return pl.pallas_call(
        _finalize_kernel,
        out_shape=[
            jax.ShapeDtypeStruct(hs.shape, f32),
            jax.ShapeDtypeStruct(hr.shape, f32),
        ],
    )(hs, hr, p0, p1, aux0, aux1, aux2)


def _make_sc_weights(n_ent, n_rel, n_edges):
    nw = NC * NS
    epw = n_edges // nw
    n_chunks = epw // CHUNK
    f32 = jnp.float32
    mesh = plsc.VectorSubcoreMesh(core_axis_name="c", subcore_axis_name="s")

    @functools.partial(
        pl.kernel,
        compiler_params=_sc_compiler_params(),
        out_type=[
            jax.ShapeDtypeStruct((n_edges,), f32),
            jax.ShapeDtypeStruct((nw, n_ent), f32),
            jax.ShapeDtypeStruct((nw, n_rel), f32),
            jax.ShapeDtypeStruct((nw, n_rel), f32),
        ],
        mesh=mesh,
        scratch_types=[
            pltpu.VMEM((CHUNK,), jnp.int32),
            pltpu.VMEM((CHUNK,), jnp.int32),
            pltpu.VMEM((CHUNK,), jnp.int32),
            pltpu.VMEM((CHUNK,), jnp.int32),
            pltpu.VMEM((CHUNK,), jnp.int32),
            pltpu.VMEM((CHUNK,), jnp.int32),
            pltpu.VMEM((CHUNK,), f32),
            pltpu.VMEM((CHUNK,), f32),
            pltpu.VMEM((n_ent,), f32),
            pltpu.VMEM((n_rel,), f32),
            pltpu.VMEM((n_ent,), f32),
            pltpu.VMEM((n_ent,), f32),
            pltpu.VMEM((n_rel,), f32),
            pltpu.VMEM((n_rel,), f32),
            pltpu.SemaphoreType.DMA,
            pltpu.SemaphoreType.DMA,
            pltpu.SemaphoreType.DMA,
            pltpu.SemaphoreType.DMA,
        ],
    )
    def sc_weights(t0_hbm, t1_hbm, t2_hbm, s0_hbm, s1_hbm, s2_hbm,
                   e_hbm, aux0_hbm, aux1_hbm, aux2_hbm,
                   i0a, i1a, i2a, i0b, i1b, i2b, ea, eb,
                   s0t, s1t, s2t, a0l, a1l, a2l,
                   semia, semib, semwa, semwb):
        c = lax.axis_index("c")
        s = lax.axis_index("s")
        wid = c * NS + s
        zero16 = jnp.zeros((L,), f32)
        one16 = jnp.ones((L,), f32)
        ebase = wid * epw

        bufs = [(i0a, i1a, i2a, ea, semia, semwa),
                (i0b, i1b, i2b, eb, semib, semwb)]

        def idx_descs(ci, B):
            i0v, i1v, i2v, _, semi, _ = bufs[B]
            off = ebase + ci * CHUNK
            return [
                pltpu.make_async_copy(t0_hbm.at[pl.ds(off, CHUNK)], i0v, semi),
                pltpu.make_async_copy(t1_hbm.at[pl.ds(off, CHUNK)], i1v, semi),
                pltpu.make_async_copy(t2_hbm.at[pl.ds(off, CHUNK)], i2v, semi),
            ]

        def wb_desc(ci, B):
            _, _, _, e80, _, semw = bufs[B]
            off = ebase + ci * CHUNK
            return pltpu.make_async_copy(e80, e_hbm.at[pl.ds(off, CHUNK)], semw)

        pltpu.sync_copy(s0_hbm, s0t)
        pltpu.sync_copy(s1_hbm, s1t)
        pltpu.sync_copy(s2_hbm, s2t)

        @pl.loop(0, n_ent // L)
        def _(q):
            a0l[pl.ds(q * L, L)] = zero16

        @pl.loop(0, n_rel // L)
        def _(q):
            a1l[pl.ds(q * L, L)] = zero16
            a2l[pl.ds(q * L, L)] = zero16

        def compute_chunk(B):
            i0v, i1v, i2v, e80, _, _ = bufs[B]

            @pl.loop(0, CHUNK // L)
            def _(q):
                sl = pl.ds(q * L, L)
                a0 = i0v[sl]
                a1 = i1v[sl]
                a2 = i2v[sl]
                d = (plsc.load_gather(s0t, [a0]) +
                     plsc.load_gather(s1t, [a1]) +
                     plsc.load_gather(s2t, [a2]))
                lk = jnp.where(d >= 0, d, d * f32(0.01))
                e = jnp.exp(-lk)
                e80[sl] = e
                plsc.addupdate_scatter(a0l, [a0], e)
                plsc.addupdate_scatter(a1l, [a1], e)
                plsc.addupdate_scatter(a2l, [a1], one16)

        for dsc in idx_descs(0, 0):
            dsc.start()
        for dsc in idx_descs(1, 1):
            dsc.start()

        @pl.loop(0, n_chunks // 2)
        def _(gg):
            for b in (0, 1):
                ci = gg * 2 + b
                for dsc in idx_descs(ci, b):
                    dsc.wait()

                @pl.when(gg >= 1)
                def _():
                    wb_desc(ci - 2, b).wait()

                compute_chunk(b)
                wb_desc(ci, b).start()

                @pl.when(ci + 2 < n_chunks)
                def _():
                    for dsc in idx_descs(ci + 2, b):
                        dsc.start()

        if n_chunks % 2 == 1:
            last = n_chunks - 1
            for dsc in idx_descs(last, 0):
                dsc.wait()
            wb_desc(last - 2, 0).wait()
            compute_chunk(0)
            wb_desc(last, 0).start()
            wb_desc(last - 1, 1).wait()
            wb_desc(last, 0).wait()
        else:
            wb_desc(n_chunks - 2, 0).wait()
            wb_desc(n_chunks - 1, 1).wait()

        pltpu.sync_copy(a0l, aux0_hbm.at[wid])
        pltpu.sync_copy(a1l, aux1_hbm.at[wid])
        pltpu.sync_copy(a2l, aux2_hbm.at[wid])

    return sc_weights


def _make_sc_agg(n_ent, n_rel, n_edges):
    edges_per_tile = n_edges // NS
    n_chunks = edges_per_tile // CHUNK
    assert n_chunks % 2 == 0
    nrb_ent = n_ent // ROWB
    nrb_rel = n_rel // ROWB
    n_max = max(n_ent, n_rel)
    f32 = jnp.float32
    mesh = plsc.VectorSubcoreMesh(core_axis_name="c", subcore_axis_name="s")

    idx_t = pltpu.VMEM((CHUNK,), jnp.int32)
    row_t = pltpu.VMEM((CHUNK, D), f32)
    e_t = pltpu.VMEM((CHUNK + L,), f32)

    @functools.partial(
        pl.kernel,
        compiler_params=_sc_compiler_params(),
        out_type=[
            jax.ShapeDtypeStruct((n_ent, D), f32),
            jax.ShapeDtypeStruct((n_rel, D), f32),
        ],
        mesh=mesh,
        scratch_types=(
            [idx_t] * 6 + [idx_t] * 2 + [row_t] * 4 + [e_t] * 2 +
            [pltpu.VMEM_SHARED((n_max, D), f32)] +
            [pltpu.SemaphoreType.DMA] * 6
        ),
    )
    def sc_agg(t0_hbm, t1_hbm, t2_hbm, e_hbm, p0_hbm, p1_hbm, p2_hbm,
               hent_hbm, hrel_hbm,
               i0a, i1a, i2a, i0b, i1b, i2b, ka, kb, gaa, gba, gab, gbb,
               ea, eb, accum,
               semia, semib, semga, semgb, semsa, semsb):
        c = lax.axis_index("c")
        s = lax.axis_index("s")
        zero16 = jnp.zeros((L,), f32)
        ebase = s * edges_per_tile

        bufs = [
            (i0a, i1a, i2a, ka, gaa, gba, ea, semia, semga, semsa),
            (i0b, i1b, i2b, kb, gab, gbb, eb, semib, semgb, semsb),
        ]

        def idx_descs(ci, B):
            i0v, i1v, i2v, _, _, _, e80, semi, _, _ = bufs[B]
            off = ebase + ci * CHUNK
            return [
                pltpu.make_async_copy(t0_hbm.at[pl.ds(off, CHUNK)], i0v, semi),
                pltpu.make_async_copy(t1_hbm.at[pl.ds(off, CHUNK)], i1v, semi),
                pltpu.make_async_copy(t2_hbm.at[pl.ds(off, CHUNK)], i2v, semi),
                pltpu.make_async_copy(e_hbm.at[pl.ds(off, CHUNK)],
                                      e80.at[pl.ds(0, CHUNK)], semi),
            ]

        def issue_idx(ci, B):
            for dsc in idx_descs(ci, B):
                dsc.start()

        def wait_idx(ci, B):
            for dsc in idx_descs(ci, B):
                dsc.wait()

        def gather_descs(core, B):
            i0v, i1v, i2v, _, gav, gbv, _, _, semg, _ = bufs[B]
            if core == 0:
                return [pltpu.make_async_copy(p1_hbm.at[i1v], gav, semg),
                        pltpu.make_async_copy(p2_hbm.at[i2v], gbv, semg)]
            return [pltpu.make_async_copy(p0_hbm.at[i0v], gav, semg),
                    pltpu.make_async_copy(p2_hbm.at[i2v], gbv, semg)]

        def for_core(fn):
            @pl.when(c == 0)
            def _():
                fn(0)

            @pl.when(c == 1)
            def _():
                fn(1)

        def issue_gather(B):
            for_core(lambda core: [d.start() for d in gather_descs(core, B)])

        def wait_gather(B):
            for_core(lambda core: [d.wait() for d in gather_descs(core, B)])

        def scatter_desc(B):
            _, _, _, kv, gav, _, _, _, _, sems = bufs[B]
            return pltpu.make_async_copy(gav, accum.at[kv], sems)

        def copy_key(B):
            i0v, i1v, _, kv, _, _, _, _, _, _ = bufs[B]

            @pl.when(c == 0)
            def _():
                for q in range(CHUNK // L):
                    sl = pl.ds(q * L, L)
                    kv[sl] = i0v[sl]

            @pl.when(c == 1)
            def _():
                for q in range(CHUNK // L):
                    sl = pl.ds(q * L, L)
                    kv[sl] = i1v[sl]

        def compute(B):
            _, _, _, _, gav, gbv, e80, _, _, _ = bufs[B]

            @pl.loop(0, CHUNK)
            def _(j):
                es = e80[pl.ds(j, L)][0]
                for k in range(D // L):
                    sl = pl.ds(k * L, L)
                    gav[j, sl] = (gav[j, sl] + gbv[j, sl]) * es

        @pl.loop(0, ROWB)
        def _(rr):
            for k in range(D // L):
                gaa[rr, pl.ds(k * L, L)] = zero16

        nrb = jnp.where(c == 0, nrb_ent, nrb_rel)

        @pl.loop(s, nrb, step=NS)
        def _(bi):
            pltpu.sync_copy(gaa, accum.at[pl.ds(bi * ROWB, ROWB)])

        plsc.subcore_barrier()

        issue_idx(0, 0)
        issue_idx(1, 1)
        wait_idx(0, 0)
        issue_gather(0)

        @pl.loop(0, n_chunks // 2)
        def _(gg):
            for b in (0, 1):
                bn = 1 - b
                ci = gg * 2 + b
                wait_gather(b)
                copy_key(b)

                @pl.when(ci + 1 < n_chunks)
                def _():
                    wait_idx(ci + 1, bn)

                    @pl.when(ci >= 1)
                    def _():
                        scatter_desc(bn).wait()
                    issue_gather(bn)

                compute(b)

                @pl.when(ci + 2 < n_chunks)
                def _():
                    issue_idx(ci + 2, b)

                pltpu.async_copy(bufs[b][4], accum.at[bufs[b][3]],
                                 bufs[b][9], add=True)

        scatter_desc(0).wait()
        scatter_desc(1).wait()

        plsc.subcore_barrier()

        @pl.when(c == 0)
        def _():
            @pl.loop(s, nrb_ent, step=NS)
            def _(bi):
                base = bi * ROWB
                pltpu.sync_copy(accum.at[pl.ds(base, ROWB)],
                                hent_hbm.at[pl.ds(base, ROWB)])

        @pl.when(c == 1)
        def _():
            @pl.loop(s, nrb_rel, step=NS)
            def _(bi):
                base = bi * ROWB
                pltpu.sync_copy(accum.at[pl.ds(base, ROWB)],
                                hrel_hbm.at[pl.ds(base, ROWB)])

    return sc_agg


def kernel(triplets, ent_embed, rel_embed, W_a, b_a, W_a2, b_a2):
    n_ent = ent_embed.shape[0]
    n_rel = rel_embed.shape[0]
    n_edges = triplets.shape[0]

    t0 = triplets[:, 0]
    t1 = triplets[:, 1]
    t2 = triplets[:, 2]

    w0t = W_a[:, 0:D].T
    w1t = W_a[:, D:2 * D].T
    w2t = W_a[:, 2 * D:3 * D].T
    ba_row = b_a.reshape(1, D)
    w2col = W_a2.T
    ba2 = b_a2.reshape(1, 1)

    p0, p1, p2, s0, s1, s2 = _build_tables(
        ent_embed, rel_embed, w0t, w1t, w2t, ba_row, w2col, ba2)

    sc_weights = _make_sc_weights(n_ent, n_rel, n_edges)
    e_all, aux0, aux1, aux2 = sc_weights(t0, t1, t2, s0, s1, s2)

    sc_agg = _make_sc_agg(n_ent, n_rel, n_edges)
    hs, hr = sc_agg(t0, t1, t2, e_all, p0, p1, p2)

    return _finalize(hs, hr, p0, p1, aux0, aux1, aux2)
